# baseline (speedup 1.0000x reference)
import functools

import jax
import jax.numpy as jnp
from jax import lax
from jax.experimental import pallas as pl
from jax.experimental.pallas import tpu as pltpu
from jax.experimental.pallas import tpu_sc as plsc

_C = 16
_N = 4096
_D = 512
_ALPHA = 0.5
_BETA = 0.5

_NW = 32
_RPW = _N // _NW
_CHUNKS = _D // 16
_PH = 4
_RPP = _RPW // _PH


def _tree_sum(vals):
    while len(vals) > 1:
        nxt = [vals[i] + vals[i + 1] for i in range(0, len(vals) - 1, 2)]
        if len(vals) % 2:
            nxt.append(vals[-1])
        vals = nxt
    return vals[0]


def _sc_main(e_hbm, lab_hbm, out_sum, out_q, out_c,
             idx_v, rows_v, lab_v, asum, qacc, cacc,
             lsem, dsem0, dsem1, dsem2, dsem3):
    cid = lax.axis_index("c")
    sid = lax.axis_index("s")
    wid = cid * 16 + sid
    base = wid * _RPW
    cp_lab = pltpu.make_async_copy(lab_hbm.at[pl.ds(base, _RPW)], lab_v, lsem)
    cp_lab.start()
    lane = lax.iota(jnp.int32, 16)
    for t in range(_RPW // 16):
        idx_v[pl.ds(t * 16, 16)] = base + t * 16 + lane
    sems = [dsem0, dsem1, dsem2, dsem3]
    cps = []
    for p in range(_PH):
        cp = pltpu.make_async_copy(
            e_hbm.at[idx_v.at[pl.ds(p * _RPP, _RPP)]],
            rows_v.at[pl.ds(p * _RPP, _RPP), :], sems[p])
        cp.start()
        cps.append(cp)

    z = jnp.zeros((16,), jnp.float32)

    def zbody(i, _):
        for t in range(16):
            asum[i, pl.ds(t * 32, 16)] = z
            asum[i, pl.ds(t * 32 + 16, 16)] = z
        return 0

    lax.fori_loop(0, _C, zbody, 0)
    for c in range(_C):
        qacc[c, :] = z
        cacc[c, :] = z

    cp_lab.wait()
    ones = jnp.ones((16,), jnp.float32)

    def gbody(g, _):
        lv = lab_v[pl.ds(g * 16, 16)]
        labs = [lv[k] for k in range(16)]
        for k in range(16):
            r = g * 16 + k
            b = labs[k]
            q = None
            for half in range(2):
                xs = [rows_v[r, pl.ds((half * 16 + j) * 16, 16)]
                      for j in range(16)]
                qh = _tree_sum([x * x for x in xs])
                for j in range(16):
                    plsc.addupdate(
                        asum.at[b, pl.ds((half * 16 + j) * 16, 16)], xs[j])
                q = qh if q is None else q + qh
            plsc.addupdate(qacc.at[b, :], q)
            plsc.addupdate(cacc.at[b, :], ones)
        return 0

    for p in range(_PH):
        cps[p].wait()
        lax.fori_loop(p * (_RPP // 16), (p + 1) * (_RPP // 16), gbody, 0)

    pltpu.sync_copy(asum, out_sum.at[wid])
    pltpu.sync_copy(qacc, out_q.at[wid])
    pltpu.sync_copy(cacc, out_c.at[wid])


_sc_call = functools.partial(
    pl.kernel,
    mesh=plsc.VectorSubcoreMesh(core_axis_name="c", subcore_axis_name="s"),
    out_type=[
        jax.ShapeDtypeStruct((_NW, _C, _D), jnp.float32),
        jax.ShapeDtypeStruct((_NW, _C, 16), jnp.float32),
        jax.ShapeDtypeStruct((_NW, _C, 16), jnp.float32),
    ],
    scratch_types=[
        pltpu.VMEM((_RPW,), jnp.int32),
        pltpu.VMEM((_RPW, _D), jnp.float32),
        pltpu.VMEM((_RPW,), jnp.int32),
        pltpu.VMEM((_C, _D), jnp.float32),
        pltpu.VMEM((_C, 16), jnp.float32),
        pltpu.VMEM((_C, 16), jnp.float32),
        pltpu.SemaphoreType.DMA,
        pltpu.SemaphoreType.DMA,
        pltpu.SemaphoreType.DMA,
        pltpu.SemaphoreType.DMA,
        pltpu.SemaphoreType.DMA,
    ],
)(_sc_main)


def _finish_body(ps_ref, pq_ref, pc_ref, o_ref):
    sums = jnp.sum(ps_ref[...], axis=0)
    s2 = jnp.sum(jnp.sum(pq_ref[...], axis=0), axis=1,
                 keepdims=True)
    counts = jnp.sum(pc_ref[...], axis=0)[:, :1]
    safe = jnp.maximum(counts, 1.0)
    p2 = jnp.sum(sums * sums, axis=1, keepdims=True)
    intra_c = (s2 - p2 / safe) / (safe * _D)
    intra = jnp.sum(jnp.where(counts > 1.0, intra_c, 0.0))
    means = sums / safe
    mnorm2 = jnp.sum(means * means)
    tot = jnp.sum(means, axis=0, keepdims=True)
    inter = (_C * mnorm2 - jnp.sum(tot * tot)) / _D
    o_ref[0, 0] = _ALPHA * intra + _BETA * inter


def kernel(embeddings, labels):
    lab_i32 = jnp.asarray(labels, jnp.int32)
    psum, pq, pc = _sc_call(embeddings, lab_i32)
    out = pl.pallas_call(
        _finish_body,
        out_shape=jax.ShapeDtypeStruct((1, 1), jnp.float32),
        in_specs=[
            pl.BlockSpec(memory_space=pltpu.VMEM),
            pl.BlockSpec(memory_space=pltpu.VMEM),
            pl.BlockSpec(memory_space=pltpu.VMEM),
        ],
        out_specs=pl.BlockSpec(memory_space=pltpu.SMEM),
    )(psum, pq, pc)
    return out[0, 0]


# speedup vs baseline: 1.3423x; 1.3423x over previous
import functools

import jax
import jax.numpy as jnp
from jax import lax
from jax.experimental import pallas as pl
from jax.experimental.pallas import tpu as pltpu
from jax.experimental.pallas import tpu_sc as plsc

_C = 16
_N = 4096
_D = 512
_ALPHA = 0.5
_BETA = 0.5

_NW = 32
_RPW = _N // _NW
_CHUNKS = _D // 16


def _tree_sum(vals):
    while len(vals) > 1:
        nxt = [vals[i] + vals[i + 1] for i in range(0, len(vals) - 1, 2)]
        if len(vals) % 2:
            nxt.append(vals[-1])
        vals = nxt
    return vals[0]


def _sc_main(e_hbm, lab_hbm, out_sum, out_q, out_c,
             idx_v, rows_v, lab_v, asum, qacc, cacc, lsem, dsem):
    cid = lax.axis_index("c")
    sid = lax.axis_index("s")
    wid = cid * 16 + sid
    base = wid * _RPW
    cp_lab = pltpu.make_async_copy(lab_hbm.at[pl.ds(base, _RPW)],
                                  lab_v.at[pl.ds(0, _RPW)], lsem)
    cp_lab.start()
    lane = lax.iota(jnp.int32, 16)

    def ibody(t, _):
        idx_v[pl.ds(t * 16, 16)] = base + t * 16 + lane
        return 0

    lax.fori_loop(0, _RPW // 16, ibody, 0)
    cp_rows = pltpu.make_async_copy(e_hbm.at[idx_v], rows_v, dsem)
    cp_rows.start()

    z = jnp.zeros((16,), jnp.float32)

    def zbody(i, _):
        asum[i >> 5, pl.ds((i & 31) * 16, 16)] = z
        return 0

    lax.fori_loop(0, _C * _CHUNKS, zbody, 0)

    def zbody2(c, _):
        qacc[c, :] = z
        cacc[c, :] = z
        return 0

    lax.fori_loop(0, _C, zbody2, 0)
    cp_lab.wait()
    cp_rows.wait()
    ones = jnp.ones((16,), jnp.float32)

    def rbody(r, _):
        lv = lab_v[pl.ds(r, 16)]
        lab = lv[0]
        xs = [rows_v[r, pl.ds(j * 16, 16)] for j in range(_CHUNKS)]
        sq = _tree_sum([x * x for x in xs])
        for j in range(_CHUNKS):
            plsc.addupdate(asum.at[lab, pl.ds(j * 16, 16)], xs[j])
        plsc.addupdate(qacc.at[lab, :], sq)
        plsc.addupdate(cacc.at[lab, :], ones)
        return 0

    lax.fori_loop(0, _RPW, rbody, 0)
    pltpu.sync_copy(asum, out_sum.at[wid])
    pltpu.sync_copy(qacc, out_q.at[wid])
    pltpu.sync_copy(cacc, out_c.at[wid])


_sc_call = functools.partial(
    pl.kernel,
    mesh=plsc.VectorSubcoreMesh(core_axis_name="c", subcore_axis_name="s"),
    out_type=[
        jax.ShapeDtypeStruct((_NW, _C, _D), jnp.float32),
        jax.ShapeDtypeStruct((_NW, _C, 16), jnp.float32),
        jax.ShapeDtypeStruct((_NW, _C, 16), jnp.float32),
    ],
    scratch_types=[
        pltpu.VMEM((_RPW,), jnp.int32),
        pltpu.VMEM((_RPW, _D), jnp.float32),
        pltpu.VMEM((_RPW + 16,), jnp.int32),
        pltpu.VMEM((_C, _D), jnp.float32),
        pltpu.VMEM((_C, 16), jnp.float32),
        pltpu.VMEM((_C, 16), jnp.float32),
        pltpu.SemaphoreType.DMA,
        pltpu.SemaphoreType.DMA,
    ],
)(_sc_main)


def _finish_body(ps_ref, pq_ref, pc_ref, o_ref):
    sums = jnp.sum(ps_ref[...], axis=0)
    s2 = jnp.sum(jnp.sum(pq_ref[...], axis=0), axis=1,
                 keepdims=True)
    counts = jnp.sum(pc_ref[...], axis=0)[:, :1]
    safe = jnp.maximum(counts, 1.0)
    p2 = jnp.sum(sums * sums, axis=1, keepdims=True)
    intra_c = (s2 - p2 / safe) / (safe * _D)
    intra = jnp.sum(jnp.where(counts > 1.0, intra_c, 0.0))
    means = sums / safe
    mnorm2 = jnp.sum(means * means)
    tot = jnp.sum(means, axis=0, keepdims=True)
    inter = (_C * mnorm2 - jnp.sum(tot * tot)) / _D
    o_ref[0, 0] = _ALPHA * intra + _BETA * inter


def kernel(embeddings, labels):
    lab_i32 = jnp.asarray(labels, jnp.int32)
    psum, pq, pc = _sc_call(embeddings, lab_i32)
    out = pl.pallas_call(
        _finish_body,
        out_shape=jax.ShapeDtypeStruct((1, 1), jnp.float32),
        in_specs=[
            pl.BlockSpec(memory_space=pltpu.VMEM),
            pl.BlockSpec(memory_space=pltpu.VMEM),
            pl.BlockSpec(memory_space=pltpu.VMEM),
        ],
        out_specs=pl.BlockSpec(memory_space=pltpu.SMEM),
    )(psum, pq, pc)
    return out[0, 0]
